# baseline (speedup 1.0000x reference)
import functools

import jax
import jax.numpy as jnp
from jax import lax
from jax.experimental import pallas as pl
from jax.experimental.pallas import tpu as pltpu
from jax.experimental.pallas import tpu_sc as plsc


def _sc_dims():
    info = plsc.get_sparse_core_info()
    return info.num_cores, info.num_subcores


def _pick_chunk(per_w, max_ch=120):
    best = None
    for ch in range(8, max_ch + 1, 8):
        if per_w % ch == 0 and (per_w // ch) % 2 == 0:
            best = ch
    assert best is not None, per_w
    return best


def _sc_gather(table, idx, d):
    nc, ns = _sc_dims()
    nw = nc * ns
    b = idx.shape[0]
    per_w = b // nw
    ch = 80
    while per_w % ch or ch % 8:
        ch //= 2
    n_ch = per_w // ch
    mesh = plsc.VectorSubcoreMesh(core_axis_name="c", subcore_axis_name="s")

    nbuf = min(3, n_ch)

    @functools.partial(
        pl.kernel,
        mesh=mesh,
        out_type=jax.ShapeDtypeStruct((b, d), jnp.float32),
        scratch_types=[pltpu.VMEM((ch,), jnp.int32)] * nbuf
        + [pltpu.VMEM((ch, d), jnp.float32)] * nbuf
        + [pltpu.SemaphoreType.DMA] * (2 * nbuf),
    )
    def k(table_hbm, idx_hbm, out_hbm, *scr):
        idx_v = scr[:nbuf]
        rows_v = scr[nbuf : 2 * nbuf]
        gsem = scr[2 * nbuf : 3 * nbuf]
        wsem = scr[3 * nbuf :]
        wid = lax.axis_index("s") * nc + lax.axis_index("c")
        base = wid * per_w

        def issue(c):
            bb = c % nbuf
            pltpu.sync_copy(idx_hbm.at[pl.ds(base + c * ch, ch)], idx_v[bb])
            pltpu.make_async_copy(
                table_hbm.at[idx_v[bb]], rows_v[bb], gsem[bb]
            ).start()

        def complete(c):
            bb = c % nbuf
            pltpu.make_async_copy(
                table_hbm.at[idx_v[bb]], rows_v[bb], gsem[bb]
            ).wait()
            pltpu.make_async_copy(
                rows_v[bb], out_hbm.at[pl.ds(base + c * ch, ch)], wsem[bb]
            ).start()

        def wait_wb(c):
            bb = c % nbuf
            pltpu.make_async_copy(
                rows_v[bb], out_hbm.at[pl.ds(base + c * ch, ch)], wsem[bb]
            ).wait()

        for c in range(n_ch):
            if c >= nbuf:
                wait_wb(c - nbuf)
            issue(c)
            if c >= 1:
                complete(c - 1)
        complete(n_ch - 1)
        for c in range(max(0, n_ch - nbuf), n_ch):
            wait_wb(c)

    return k(table, idx)


def _sc_scatter4(parts, idx, n_out, d):
    nc, ns = _sc_dims()
    nw = nc * ns
    bp = parts[0].shape[0]
    n_parts = len(parts)
    b = bp * n_parts
    per_w = b // nw
    assert bp % per_w == 0
    w_per_part = nw // n_parts
    ch = _pick_chunk(per_w)
    n_ch = per_w // ch
    mesh = plsc.VectorSubcoreMesh(core_axis_name="c", subcore_axis_name="s")

    nbuf = min(3, n_ch)

    @functools.partial(
        pl.kernel,
        mesh=mesh,
        out_type=jax.ShapeDtypeStruct((n_out, d), jnp.float32),
        scratch_types=[pltpu.VMEM((ch,), jnp.int32)] * nbuf
        + [pltpu.VMEM((ch, d), jnp.float32)] * nbuf
        + [pltpu.SemaphoreType.DMA] * (2 * nbuf),
    )
    def k(*refs):
        rows_hbm = refs[:n_parts]
        idx_hbm = refs[n_parts]
        out_hbm = refs[n_parts + 1]
        scr = refs[n_parts + 2 :]
        idx_v = scr[:nbuf]
        rows_v = scr[nbuf : 2 * nbuf]
        lsem = scr[2 * nbuf : 3 * nbuf]
        ssem = scr[3 * nbuf :]
        wid = lax.axis_index("s") * nc + lax.axis_index("c")
        gbase = wid * per_w

        def run(src, lbase):
            def issue(c):
                bb = c % nbuf
                off = c * ch
                pltpu.sync_copy(
                    idx_hbm.at[pl.ds(gbase + off, ch)], idx_v[bb]
                )
                pltpu.make_async_copy(
                    src.at[pl.ds(lbase + off, ch)], rows_v[bb], lsem[bb]
                ).start()

            def complete(c):
                bb = c % nbuf
                off = c * ch
                pltpu.make_async_copy(
                    src.at[pl.ds(lbase + off, ch)], rows_v[bb], lsem[bb]
                ).wait()
                pltpu.make_async_copy(
                    rows_v[bb], out_hbm.at[idx_v[bb]], ssem[bb]
                ).start()

            def wait_store(c):
                bb = c % nbuf
                pltpu.make_async_copy(
                    rows_v[bb], out_hbm.at[idx_v[bb]], ssem[bb]
                ).wait()

            for c in range(n_ch):
                if c >= nbuf:
                    wait_store(c - nbuf)
                issue(c)
                if c >= 1:
                    complete(c - 1)
            complete(n_ch - 1)
            for c in range(max(0, n_ch - nbuf), n_ch):
                wait_store(c)

        for p in range(n_parts):
            @pl.when(wid // w_per_part == p)
            def _(p=p):
                run(rows_hbm[p], gbase - p * bp)

    return k(*parts, idx)


def _mlp_body(x_ref, w1_ref, b1_ref, w2_ref, b2_ref, w3_ref, b3_ref, o_ref):
    x = x_ref[...]
    h = jnp.dot(x, w1_ref[0], preferred_element_type=jnp.float32) + b1_ref[0]
    h = h * jax.nn.sigmoid(h)
    h = jnp.dot(h, w2_ref[0], preferred_element_type=jnp.float32) + b2_ref[0]
    h = h * jax.nn.sigmoid(h)
    o_ref[...] = (
        jnp.dot(h, w3_ref[0], preferred_element_type=jnp.float32) + b3_ref[0]
    )


def _mlp_one(x, W1, b1r, W2, b2r, W3, b3r, s):
    d_in = W1.shape[1]
    d_h = W1.shape[2]
    d_out = W3.shape[2]
    rows = x.shape[0]
    br = 1280
    nr = rows // br
    return pl.pallas_call(
        _mlp_body,
        grid=(nr,),
        in_specs=[
            pl.BlockSpec((br, d_in), lambda r: (r, 0)),
            pl.BlockSpec((1, d_in, d_h), lambda r, s=s: (s, 0, 0)),
            pl.BlockSpec((1, 1, d_h), lambda r, s=s: (s, 0, 0)),
            pl.BlockSpec((1, d_h, d_h), lambda r, s=s: (s, 0, 0)),
            pl.BlockSpec((1, 1, d_h), lambda r, s=s: (s, 0, 0)),
            pl.BlockSpec((1, d_h, d_out), lambda r, s=s: (s, 0, 0)),
            pl.BlockSpec((1, 1, d_out), lambda r, s=s: (s, 0, 0)),
        ],
        out_specs=pl.BlockSpec((br, d_out), lambda r: (r, 0)),
        out_shape=jax.ShapeDtypeStruct((rows, d_out), jnp.float32),
    )(x, W1, b1r, W2, b2r, W3, b3r)


def kernel(species, embedding, idx_H, idx_C, idx_N, idx_O, W1, b1, W2, b2, W3, b3):
    n_atoms = species.shape[0]
    d_in = embedding.shape[1]
    d_h = W1.shape[2]
    d_out = W3.shape[2]
    n_species = W1.shape[0]
    idxs = [idx_H, idx_C, idx_N, idx_O]
    per = idx_H.shape[0]
    pad_to = -(-per // 512) * 512
    pad = pad_to - per

    idx_parts = [
        jnp.pad(i.astype(jnp.int32), (0, pad), mode="edge") for i in idxs
    ]
    idx_p = jnp.concatenate(idx_parts)

    b1r = b1.reshape(n_species, 1, d_h)
    b2r = b2.reshape(n_species, 1, d_h)
    b3r = b3.reshape(n_species, 1, d_out)

    o_parts = []
    for s in range(n_species):
        x_s = _sc_gather(embedding, idx_parts[s], d_in)
        o_parts.append(_mlp_one(x_s, W1, b1r, W2, b2r, W3, b3r, s))

    return _sc_scatter4(o_parts, idx_p, n_atoms, d_out)


# speedup vs baseline: 1.0033x; 1.0033x over previous
import functools

import jax
import jax.numpy as jnp
from jax import lax
from jax.experimental import pallas as pl
from jax.experimental.pallas import tpu as pltpu
from jax.experimental.pallas import tpu_sc as plsc


def _sc_dims():
    info = plsc.get_sparse_core_info()
    return info.num_cores, info.num_subcores


def _pick_chunk(per_w, max_ch=120):
    best = None
    for ch in range(8, max_ch + 1, 8):
        if per_w % ch == 0 and (per_w // ch) % 2 == 0:
            best = ch
    assert best is not None, per_w
    return best


def _sc_gather(table, idx, d):
    nc, ns = _sc_dims()
    nw = nc * ns
    b = idx.shape[0]
    per_w = b // nw
    ch = 80
    while per_w % ch or ch % 8:
        ch //= 2
    n_ch = per_w // ch
    mesh = plsc.VectorSubcoreMesh(core_axis_name="c", subcore_axis_name="s")

    nbuf = min(3, n_ch)

    @functools.partial(
        pl.kernel,
        mesh=mesh,
        out_type=jax.ShapeDtypeStruct((b, d), jnp.float32),
        scratch_types=[pltpu.VMEM((ch,), jnp.int32)] * nbuf
        + [pltpu.VMEM((ch, d), jnp.float32)] * nbuf
        + [pltpu.SemaphoreType.DMA] * (2 * nbuf),
    )
    def k(table_hbm, idx_hbm, out_hbm, *scr):
        idx_v = scr[:nbuf]
        rows_v = scr[nbuf : 2 * nbuf]
        gsem = scr[2 * nbuf : 3 * nbuf]
        wsem = scr[3 * nbuf :]
        wid = lax.axis_index("s") * nc + lax.axis_index("c")
        base = wid * per_w

        def issue(c):
            bb = c % nbuf
            pltpu.sync_copy(idx_hbm.at[pl.ds(base + c * ch, ch)], idx_v[bb])
            pltpu.make_async_copy(
                table_hbm.at[idx_v[bb]], rows_v[bb], gsem[bb]
            ).start()

        def complete(c):
            bb = c % nbuf
            pltpu.make_async_copy(
                table_hbm.at[idx_v[bb]], rows_v[bb], gsem[bb]
            ).wait()
            pltpu.make_async_copy(
                rows_v[bb], out_hbm.at[pl.ds(base + c * ch, ch)], wsem[bb]
            ).start()

        def wait_wb(c):
            bb = c % nbuf
            pltpu.make_async_copy(
                rows_v[bb], out_hbm.at[pl.ds(base + c * ch, ch)], wsem[bb]
            ).wait()

        for c in range(n_ch):
            if c >= nbuf:
                wait_wb(c - nbuf)
            issue(c)
            if c >= 1:
                complete(c - 1)
        complete(n_ch - 1)
        for c in range(max(0, n_ch - nbuf), n_ch):
            wait_wb(c)

    return k(table, idx)


def _sc_scatter4(parts, idx, n_out, d):
    nc, ns = _sc_dims()
    nw = nc * ns
    bp = parts[0].shape[0]
    n_parts = len(parts)
    b = bp * n_parts
    per_w = b // nw
    assert bp % per_w == 0
    w_per_part = nw // n_parts
    ch = _pick_chunk(per_w)
    n_ch = per_w // ch
    mesh = plsc.VectorSubcoreMesh(core_axis_name="c", subcore_axis_name="s")

    nbuf = min(3, n_ch)

    @functools.partial(
        pl.kernel,
        mesh=mesh,
        out_type=jax.ShapeDtypeStruct((n_out, d), jnp.float32),
        scratch_types=[pltpu.VMEM((ch,), jnp.int32)] * nbuf
        + [pltpu.VMEM((ch, d), jnp.float32)] * nbuf
        + [pltpu.SemaphoreType.DMA] * (2 * nbuf),
    )
    def k(*refs):
        rows_hbm = refs[:n_parts]
        idx_hbm = refs[n_parts]
        out_hbm = refs[n_parts + 1]
        scr = refs[n_parts + 2 :]
        idx_v = scr[:nbuf]
        rows_v = scr[nbuf : 2 * nbuf]
        lsem = scr[2 * nbuf : 3 * nbuf]
        ssem = scr[3 * nbuf :]
        wid = lax.axis_index("s") * nc + lax.axis_index("c")
        gbase = wid * per_w

        def run(src, lbase):
            def issue(c):
                bb = c % nbuf
                off = c * ch
                pltpu.sync_copy(
                    idx_hbm.at[pl.ds(gbase + off, ch)], idx_v[bb]
                )
                pltpu.make_async_copy(
                    src.at[pl.ds(lbase + off, ch)], rows_v[bb], lsem[bb]
                ).start()

            def complete(c):
                bb = c % nbuf
                off = c * ch
                pltpu.make_async_copy(
                    src.at[pl.ds(lbase + off, ch)], rows_v[bb], lsem[bb]
                ).wait()
                pltpu.make_async_copy(
                    rows_v[bb], out_hbm.at[idx_v[bb]], ssem[bb]
                ).start()

            def wait_store(c):
                bb = c % nbuf
                pltpu.make_async_copy(
                    rows_v[bb], out_hbm.at[idx_v[bb]], ssem[bb]
                ).wait()

            for c in range(n_ch):
                if c >= nbuf:
                    wait_store(c - nbuf)
                issue(c)
                if c >= 1:
                    complete(c - 1)
            complete(n_ch - 1)
            for c in range(max(0, n_ch - nbuf), n_ch):
                wait_store(c)

        for p in range(n_parts):
            @pl.when(wid // w_per_part == p)
            def _(p=p):
                run(rows_hbm[p], gbase - p * bp)

    return k(*parts, idx)


def _mlp_body(x_ref, w1_ref, b1_ref, w2_ref, b2_ref, w3_ref, b3_ref, o_ref):
    x = x_ref[...]
    h = jnp.dot(x, w1_ref[0], preferred_element_type=jnp.float32) + b1_ref[0]
    h = h * jax.nn.sigmoid(h)
    h = jnp.dot(h, w2_ref[0], preferred_element_type=jnp.float32) + b2_ref[0]
    h = h * jax.nn.sigmoid(h)
    o_ref[...] = (
        jnp.dot(h, w3_ref[0], preferred_element_type=jnp.float32) + b3_ref[0]
    )


def _mlp_one(x, W1, b1r, W2, b2r, W3, b3r, s):
    d_in = W1.shape[1]
    d_h = W1.shape[2]
    d_out = W3.shape[2]
    rows = x.shape[0]
    br = 640
    nr = rows // br
    return pl.pallas_call(
        _mlp_body,
        grid=(nr,),
        in_specs=[
            pl.BlockSpec((br, d_in), lambda r: (r, 0)),
            pl.BlockSpec((1, d_in, d_h), lambda r, s=s: (s, 0, 0)),
            pl.BlockSpec((1, 1, d_h), lambda r, s=s: (s, 0, 0)),
            pl.BlockSpec((1, d_h, d_h), lambda r, s=s: (s, 0, 0)),
            pl.BlockSpec((1, 1, d_h), lambda r, s=s: (s, 0, 0)),
            pl.BlockSpec((1, d_h, d_out), lambda r, s=s: (s, 0, 0)),
            pl.BlockSpec((1, 1, d_out), lambda r, s=s: (s, 0, 0)),
        ],
        out_specs=pl.BlockSpec((br, d_out), lambda r: (r, 0)),
        out_shape=jax.ShapeDtypeStruct((rows, d_out), jnp.float32),
    )(x, W1, b1r, W2, b2r, W3, b3r)


def kernel(species, embedding, idx_H, idx_C, idx_N, idx_O, W1, b1, W2, b2, W3, b3):
    n_atoms = species.shape[0]
    d_in = embedding.shape[1]
    d_h = W1.shape[2]
    d_out = W3.shape[2]
    n_species = W1.shape[0]
    idxs = [idx_H, idx_C, idx_N, idx_O]
    per = idx_H.shape[0]
    pad_to = -(-per // 512) * 512
    pad = pad_to - per

    idx_parts = [
        jnp.pad(i.astype(jnp.int32), (0, pad), mode="edge") for i in idxs
    ]
    idx_p = jnp.concatenate(idx_parts)

    b1r = b1.reshape(n_species, 1, d_h)
    b2r = b2.reshape(n_species, 1, d_h)
    b3r = b3.reshape(n_species, 1, d_out)

    o_parts = []
    for s in range(n_species):
        x_s = _sc_gather(embedding, idx_parts[s], d_in)
        o_parts.append(_mlp_one(x_s, W1, b1r, W2, b2r, W3, b3r, s))

    return _sc_scatter4(o_parts, idx_p, n_atoms, d_out)
